# baseline (speedup 1.0000x reference)
import functools
import numpy as np
import jax
import jax.numpy as jnp
from jax import lax
from jax.experimental import pallas as pl
from jax.experimental.pallas import tpu as pltpu
from jax.experimental.pallas import tpu_sc as plsc

_GRID = 0.5
_N = 48
_VDW = 1.52
_MULTI = 1.5
_WEIGHT = 25.0
_B = _MULTI * _VDW
_NATOMS = 20000


def _sq_threshold() -> np.float32:
    r = np.float32(_MULTI * _VDW)
    c = np.float32(r) * np.float32(r)
    while np.float32(np.sqrt(np.nextafter(c, np.float32(0.0), dtype=np.float32))) >= r:
        c = np.nextafter(c, np.float32(0.0), dtype=np.float32)
    while np.float32(np.sqrt(c)) < r:
        c = np.nextafter(c, np.float32(np.inf), dtype=np.float32)
    return c


_C = float(_sq_threshold())


def _sc_kernel(vx_hbm, vy_hbm, vz_hbm, out_hbm, vxr, vyr, vzr, slabr, ob0r, ob1r):
    w = lax.axis_index("s") * 2 + lax.axis_index("c")
    p0 = w
    p1 = w + 32

    pltpu.sync_copy(vx_hbm, vxr)
    pltpu.sync_copy(vy_hbm, vyr)
    pltpu.sync_copy(vz_hbm, vzr)

    zero16 = jnp.zeros((16,), jnp.float32)

    def zbody(i, _):
        slabr[pl.ds(i * 16, 16)] = zero16
        return ()

    lax.fori_loop(0, 512, zbody, (), unroll=False)

    iotaf = lax.broadcasted_iota(jnp.int32, (16,), 0).astype(jnp.float32)

    def abody(t, _):
        base = t * 16
        vxv = vxr[pl.ds(base, 16)]
        vyv = vyr[pl.ds(base, 16)]
        vzv = vzr[pl.ds(base, 16)]
        minxv = jnp.maximum(0, ((vxv - _B) / _GRID).astype(jnp.int32))
        minyv = jnp.maximum(0, ((vyv - _B) / _GRID).astype(jnp.int32))
        minzv = jnp.maximum(0, ((vzv - _B) / _GRID).astype(jnp.int32))
        minxs = [minxv[l] for l in range(16)]

        for l in range(16):
            minx = minxs[l]
            h0 = (minx <= p0) & (p0 < minx + 12)
            h1 = (minx <= p1) & (p1 < minx + 12) & (p1 < _N)

            @pl.when(h0 | h1)
            def _(l=l, h0=h0, h1=h1):
                vx = vxv[l]
                vy = vyv[l]
                vz = vzv[l]
                miny = minyv[l]
                minz = minzv[l]
                zf = (minz.astype(jnp.float32) + iotaf) * _GRID
                dzv = vz - zf
                dz2v = dzv * dzv

                def do_plane(p, slot, hit):
                    @pl.when(hit)
                    def _():
                        dx = vx - _GRID * p.astype(jnp.float32)
                        dx2 = dx * dx
                        for j in range(12):
                            y = miny + j
                            dy = vy - _GRID * y.astype(jnp.float32)
                            s = dx2 + dy * dy
                            d2 = s + dz2v
                            val = jnp.where(d2 < _C, 1.0, 0.0).astype(jnp.float32)
                            addr = slot * 4096 + y * 64 + minz
                            plsc.addupdate(slabr.at[pl.ds(addr, 16)], val)

                do_plane(p0, 0, h0)
                do_plane(p1, 1, h1)

        return ()

    lax.fori_loop(0, _NATOMS // 16, abody, (), unroll=False)

    def emit_plane(p, slot, active):
        @pl.when(active)
        def _():
            def ybody(y, _):
                for c in range(3):
                    cnt = slabr[pl.ds(slot * 4096 + y * 64 + c * 16, 16)]
                    cov = cnt >= 0.9
                    ob0r[pl.ds(y * 48 + c * 16, 16)] = jnp.where(cov, 1.0, 0.0).astype(jnp.float32)
                    ob1r[pl.ds(y * 48 + c * 16, 16)] = jnp.where(cov, _WEIGHT, 1.0).astype(jnp.float32)
                return ()

            lax.fori_loop(0, _N, ybody, (), unroll=False)
            pltpu.sync_copy(ob0r, out_hbm.at[0, p])
            pltpu.sync_copy(ob1r, out_hbm.at[1, p])

    emit_plane(p0, 0, jnp.bool_(True))
    emit_plane(p1, 1, p1 < _N)


def kernel(vecs):
    vx = vecs[:, 0]
    vy = vecs[:, 1]
    vz = vecs[:, 2]
    mesh = plsc.VectorSubcoreMesh(core_axis_name="c", subcore_axis_name="s")
    k = functools.partial(
        pl.kernel,
        mesh=mesh,
        out_type=jax.ShapeDtypeStruct((2, _N, _N * _N), jnp.float32),
        scratch_types=[
            pltpu.VMEM((_NATOMS,), jnp.float32),
            pltpu.VMEM((_NATOMS,), jnp.float32),
            pltpu.VMEM((_NATOMS,), jnp.float32),
            pltpu.VMEM((8192,), jnp.float32),
            pltpu.VMEM((_N * _N,), jnp.float32),
            pltpu.VMEM((_N * _N,), jnp.float32),
        ],
    )(_sc_kernel)
    out = k(vx, vy, vz)
    return out.reshape(2, _N, _N, _N)


# speedup vs baseline: 3.1019x; 3.1019x over previous
import functools
import numpy as np
import jax
import jax.numpy as jnp
from jax import lax
from jax.experimental import pallas as pl
from jax.experimental.pallas import tpu as pltpu
from jax.experimental.pallas import tpu_sc as plsc

_GRID = 0.5
_N = 48
_VDW = 1.52
_MULTI = 1.5
_WEIGHT = 25.0
_B = _MULTI * _VDW
_NATOMS = 20000


def _sq_threshold() -> np.float32:
    r = np.float32(_MULTI * _VDW)
    c = np.float32(r) * np.float32(r)
    while np.float32(np.sqrt(np.nextafter(c, np.float32(0.0), dtype=np.float32))) >= r:
        c = np.nextafter(c, np.float32(0.0), dtype=np.float32)
    while np.float32(np.sqrt(c)) < r:
        c = np.nextafter(c, np.float32(np.inf), dtype=np.float32)
    return c


_C = float(_sq_threshold())


def _sc_kernel(vx_hbm, vy_hbm, vz_hbm, out_hbm, vxr, vyr, vzr, slabr, ob0r, ob1r):
    w = lax.axis_index("s") * 2 + lax.axis_index("c")
    p0 = w
    p1 = w + 32

    pltpu.sync_copy(vx_hbm, vxr)
    pltpu.sync_copy(vy_hbm, vyr)
    pltpu.sync_copy(vz_hbm, vzr)

    zero16 = jnp.zeros((16,), jnp.float32)

    def zbody(i, _):
        slabr[pl.ds(i * 16, 16)] = zero16
        return ()

    lax.fori_loop(0, 512, zbody, (), unroll=False)

    iotaf = lax.broadcasted_iota(jnp.int32, (16,), 0).astype(jnp.float32)

    def abody(t, _):
        base = t * 16
        vxv = vxr[pl.ds(base, 16)]
        vyv = vyr[pl.ds(base, 16)]
        vzv = vzr[pl.ds(base, 16)]
        minxv = jnp.maximum(0, ((vxv - _B) / _GRID).astype(jnp.int32))
        minyv = jnp.maximum(0, ((vyv - _B) / _GRID).astype(jnp.int32))
        minzv = jnp.maximum(0, ((vzv - _B) / _GRID).astype(jnp.int32))
        minxs = [minxv[l] for l in range(16)]

        for l in range(16):
            minx = minxs[l]
            h0 = (minx <= p0) & (p0 < minx + 12)
            h1 = (minx <= p1) & (p1 < minx + 12) & (p1 < _N)

            @pl.when(h0 | h1)
            def _(l=l, h0=h0):
                vx = vxv[l]
                vy = vyv[l]
                vz = vzv[l]
                miny = minyv[l]
                minz = minzv[l]
                zf = (minz.astype(jnp.float32) + iotaf) * _GRID
                dzv = vz - zf
                dz2v = dzv * dzv

                p = jnp.where(h0, p0, p1)
                slot_base = jnp.where(h0, 0, 4096)
                dx = vx - _GRID * p.astype(jnp.float32)
                dx2 = dx * dx
                for j in range(12):
                    y = miny + j
                    dy = vy - _GRID * y.astype(jnp.float32)
                    s = dx2 + dy * dy
                    d2 = s + dz2v
                    val = jnp.where(d2 < _C, 1.0, 0.0).astype(jnp.float32)
                    addr = slot_base + y * 64 + minz
                    plsc.addupdate(slabr.at[pl.ds(addr, 16)], val)

        return ()

    lax.fori_loop(0, _NATOMS // 16, abody, (), unroll=False)

    def emit_plane(p, slot, active):
        @pl.when(active)
        def _():
            def ybody(y, _):
                for c in range(3):
                    cnt = slabr[pl.ds(slot * 4096 + y * 64 + c * 16, 16)]
                    cov = cnt >= 0.9
                    ob0r[pl.ds(y * 48 + c * 16, 16)] = jnp.where(cov, 1.0, 0.0).astype(jnp.float32)
                    ob1r[pl.ds(y * 48 + c * 16, 16)] = jnp.where(cov, _WEIGHT, 1.0).astype(jnp.float32)
                return ()

            lax.fori_loop(0, _N, ybody, (), unroll=False)
            pltpu.sync_copy(ob0r, out_hbm.at[0, p])
            pltpu.sync_copy(ob1r, out_hbm.at[1, p])

    emit_plane(p0, 0, jnp.bool_(True))
    emit_plane(p1, 1, p1 < _N)


def kernel(vecs):
    vx = vecs[:, 0]
    vy = vecs[:, 1]
    vz = vecs[:, 2]
    mesh = plsc.VectorSubcoreMesh(core_axis_name="c", subcore_axis_name="s")
    k = functools.partial(
        pl.kernel,
        mesh=mesh,
        out_type=jax.ShapeDtypeStruct((2, _N, _N * _N), jnp.float32),
        scratch_types=[
            pltpu.VMEM((_NATOMS,), jnp.float32),
            pltpu.VMEM((_NATOMS,), jnp.float32),
            pltpu.VMEM((_NATOMS,), jnp.float32),
            pltpu.VMEM((8192,), jnp.float32),
            pltpu.VMEM((_N * _N,), jnp.float32),
            pltpu.VMEM((_N * _N,), jnp.float32),
        ],
    )(_sc_kernel)
    out = k(vx, vy, vz)
    return out.reshape(2, _N, _N, _N)
